# Initial kernel scaffold; baseline (speedup 1.0000x reference)
import functools

import jax
import jax.numpy as jnp
from jax import lax
from jax.experimental import pallas as pl
from jax.experimental.pallas import tpu as pltpu
from jax.experimental.pallas import tpu_sc as plsc

BATCH = 4096
SEQ = 200
D = 64

NC = 2
NS = 16
NW = NC * NS
L = 16

TOTAL = BATCH * SEQ
PER_W = TOTAL // NW
CHUNK = 128
N_CHUNKS = PER_W // CHUNK
NBUF = 4
VPR = D // L


def _add_pos(rows_v, pos2_v, b, k):
    base = lax.rem(k * CHUNK, SEQ)

    def body(r, carry):
        pr = base + r
        for c in range(VPR):
            x = pos2_v[pr, pl.ds(c * L, L)]
            plsc.addupdate(rows_v.at[b, r, pl.ds(c * L, L)], x)
        return carry

    lax.fori_loop(0, CHUNK, body, 0)


def _kernel_body(ids_hbm, tok_hbm, pos_hbm, out_hbm,
                 idx_v, pos2_v, rows_v, gs0, gs1, gs2, gs3,
                 ss0, ss1, ss2, ss3):
    gsem = (gs0, gs1, gs2, gs3)
    ssem = (ss0, ss1, ss2, ss3)
    wid = lax.axis_index("s") * NC + lax.axis_index("c")
    idx_row0 = wid * N_CHUNKS
    out_row0 = wid * PER_W

    pltpu.sync_copy(ids_hbm.at[pl.ds(idx_row0, N_CHUNKS)], idx_v)
    pltpu.sync_copy(pos_hbm, pos2_v.at[pl.ds(0, SEQ)])
    pltpu.sync_copy(pos_hbm, pos2_v.at[pl.ds(SEQ, SEQ)])

    def start_gather(k, b):
        pltpu.make_async_copy(
            tok_hbm.at[idx_v.at[k]], rows_v.at[b], gsem[b]).start()

    def wait_gather(b):
        pltpu.make_async_copy(
            tok_hbm.at[idx_v.at[0]], rows_v.at[b], gsem[b]).wait()

    def start_store(k, b):
        pltpu.make_async_copy(
            rows_v.at[b], out_hbm.at[pl.ds(out_row0 + k * CHUNK, CHUNK)],
            ssem[b]).start()

    def wait_store(b):
        pltpu.make_async_copy(
            rows_v.at[b], out_hbm.at[pl.ds(out_row0, CHUNK)], ssem[b]).wait()

    start_gather(0, 0)
    start_gather(1, 1)
    for k in (0, 1):
        start_gather(k + 2, k + 2)
        wait_gather(k)
        _add_pos(rows_v, pos2_v, k, k)
        start_store(k, k)

    def group(g, carry):
        for j in range(NBUF):
            k = 2 + g * NBUF + j
            b = (2 + j) % NBUF
            wait_store(j)
            start_gather(k + 2, j)
            wait_gather(b)
            _add_pos(rows_v, pos2_v, b, k)
            start_store(k, b)
        return carry

    lax.fori_loop(0, (N_CHUNKS - NBUF) // NBUF, group, 0)

    for k in (N_CHUNKS - 2, N_CHUNKS - 1):
        b = k % NBUF
        wait_store(b - 2)
        wait_gather(b)
        _add_pos(rows_v, pos2_v, b, k)
        start_store(k, b)
    wait_store(NBUF - 2)
    wait_store(NBUF - 1)


def kernel(input_ids, token_table, pos_table):
    ids = input_ids.reshape(TOTAL // CHUNK, CHUNK).astype(jnp.int32)
    tok = token_table.astype(jnp.float32)
    pos = pos_table.astype(jnp.float32)

    mesh = plsc.VectorSubcoreMesh(core_axis_name="c", subcore_axis_name="s")
    run = pl.kernel(
        _kernel_body,
        out_type=jax.ShapeDtypeStruct((TOTAL, D), jnp.float32),
        mesh=mesh,
        scratch_types=[
            pltpu.VMEM((N_CHUNKS, CHUNK), jnp.int32),
            pltpu.VMEM((2 * SEQ, D), jnp.float32),
            pltpu.VMEM((NBUF, CHUNK, D), jnp.float32),
            pltpu.SemaphoreType.DMA, pltpu.SemaphoreType.DMA,
            pltpu.SemaphoreType.DMA, pltpu.SemaphoreType.DMA,
            pltpu.SemaphoreType.DMA, pltpu.SemaphoreType.DMA,
            pltpu.SemaphoreType.DMA, pltpu.SemaphoreType.DMA,
        ],
    )
    out = run(ids, tok, pos)
    return out.reshape(BATCH, SEQ, D)


# speedup vs baseline: 3.0051x; 3.0051x over previous
import functools

import jax
import jax.numpy as jnp
from jax import lax
from jax.experimental import pallas as pl
from jax.experimental.pallas import tpu as pltpu
from jax.experimental.pallas import tpu_sc as plsc

BATCH = 4096
SEQ = 200
D = 64

NC = 2
NS = 16
NW = NC * NS
L = 16

TOTAL = BATCH * SEQ
PER_W = TOTAL // NW
CHUNK = 128
N_CHUNKS = PER_W // CHUNK
NBUF = 4
VPR = D // L


def _add_pos(rows_v, pos2_v, b, k):
    base = lax.rem(k * CHUNK, SEQ)

    def body(r, carry):
        pr = base + r
        for c in range(VPR):
            x = pos2_v[pr, pl.ds(c * L, L)]
            plsc.addupdate(rows_v.at[b, r, pl.ds(c * L, L)], x)
        return carry

    lax.fori_loop(0, CHUNK, body, 0)


def _kernel_body(ids_hbm, tok_hbm, pos_hbm, out_hbm,
                 idx_v, pos2_v, rows_v, gs0, gs1, gs2, gs3,
                 ss0, ss1, ss2, ss3):
    gsem = (gs0, gs1, gs2, gs3)
    ssem = (ss0, ss1, ss2, ss3)
    wid = lax.axis_index("s") * NC + lax.axis_index("c")
    idx_row0 = wid * N_CHUNKS
    out_row0 = wid * PER_W

    pltpu.sync_copy(ids_hbm.at[pl.ds(idx_row0, N_CHUNKS)], idx_v)
    pltpu.sync_copy(pos_hbm, pos2_v.at[pl.ds(0, SEQ)])
    pltpu.sync_copy(pos_hbm, pos2_v.at[pl.ds(SEQ, SEQ)])

    def start_gather(k, b):
        pltpu.make_async_copy(
            tok_hbm.at[idx_v.at[k]], rows_v.at[b], gsem[b]).start()

    def wait_gather(b):
        pltpu.make_async_copy(
            tok_hbm.at[idx_v.at[0]], rows_v.at[b], gsem[b]).wait()

    def start_store(k, b):
        pltpu.make_async_copy(
            rows_v.at[b], out_hbm.at[pl.ds(out_row0 + k * CHUNK, CHUNK)],
            ssem[b]).start()

    def wait_store(b):
        pltpu.make_async_copy(
            rows_v.at[b], out_hbm.at[pl.ds(out_row0, CHUNK)], ssem[b]).wait()

    start_gather(0, 0)
    start_gather(1, 1)
    for k in (0, 1):
        start_gather(k + 2, k + 2)
        wait_gather(k)
        _add_pos(rows_v, pos2_v, k, k)
        start_store(k, k)

    def group(g, carry):
        for j in range(NBUF):
            k = 2 + g * NBUF + j
            b = (2 + j) % NBUF
            wait_store(j)
            start_gather(k + 2, j)
            wait_gather(b)
            _add_pos(rows_v, pos2_v, b, k)
            start_store(k, b)
        return carry

    lax.fori_loop(0, (N_CHUNKS - NBUF) // NBUF, group, 0)

    for k in (N_CHUNKS - 2, N_CHUNKS - 1):
        b = k % NBUF
        wait_store(b - 2)
        wait_gather(b)
        _add_pos(rows_v, pos2_v, b, k)
        start_store(k, b)
    wait_store(NBUF - 2)
    wait_store(NBUF - 1)


def kernel(input_ids, token_table, pos_table):
    ids = input_ids.reshape(TOTAL // CHUNK, CHUNK).astype(jnp.int32)
    tok = token_table.astype(jnp.float32)
    pos = pos_table.astype(jnp.float32)

    mesh = plsc.VectorSubcoreMesh(core_axis_name="c", subcore_axis_name="s")
    run = pl.kernel(
        _kernel_body,
        out_type=jax.ShapeDtypeStruct((TOTAL, D), jnp.float32),
        mesh=mesh,
        compiler_params=pltpu.CompilerParams(use_tc_tiling_on_sc=False),
        scratch_types=[
            pltpu.VMEM((N_CHUNKS, CHUNK), jnp.int32),
            pltpu.VMEM((2 * SEQ, D), jnp.float32),
            pltpu.VMEM((NBUF, CHUNK, D), jnp.float32),
            pltpu.SemaphoreType.DMA, pltpu.SemaphoreType.DMA,
            pltpu.SemaphoreType.DMA, pltpu.SemaphoreType.DMA,
            pltpu.SemaphoreType.DMA, pltpu.SemaphoreType.DMA,
            pltpu.SemaphoreType.DMA, pltpu.SemaphoreType.DMA,
        ],
    )
    out = run(ids, tok, pos)
    return out.reshape(BATCH, SEQ, D)


# speedup vs baseline: 4.1765x; 1.3898x over previous
import functools

import jax
import jax.numpy as jnp
from jax import lax
from jax.experimental import pallas as pl
from jax.experimental.pallas import tpu as pltpu
from jax.experimental.pallas import tpu_sc as plsc

BATCH = 4096
SEQ = 200
D = 64

NC = 2
NS = 16
NW = NC * NS
L = 16

TOTAL = BATCH * SEQ
PER_W = TOTAL // NW
CHUNK = 128
N_CHUNKS = PER_W // CHUNK
NBUF = 4
VPR = D // L


def _add_pos(rows_v, pos2_v, b, k):
    base = lax.rem(k * CHUNK, SEQ)

    @plsc.parallel_loop(0, CHUNK, unroll=8)
    def body(r):
        pr = base + r
        for c in range(VPR):
            x = pos2_v[pr, pl.ds(c * L, L)]
            plsc.addupdate(rows_v.at[b, r, pl.ds(c * L, L)], x)


def _kernel_body(ids_hbm, tok_hbm, pos_hbm, out_hbm,
                 idx_v, pos2_v, rows_v, gs0, gs1, gs2, gs3,
                 ss0, ss1, ss2, ss3):
    gsem = (gs0, gs1, gs2, gs3)
    ssem = (ss0, ss1, ss2, ss3)
    wid = lax.axis_index("s") * NC + lax.axis_index("c")
    idx_row0 = wid * N_CHUNKS
    out_row0 = wid * PER_W

    pltpu.sync_copy(ids_hbm.at[pl.ds(idx_row0, N_CHUNKS)], idx_v)
    pltpu.sync_copy(pos_hbm, pos2_v.at[pl.ds(0, SEQ)])
    pltpu.sync_copy(pos_hbm, pos2_v.at[pl.ds(SEQ, SEQ)])

    def start_gather(k, b):
        pltpu.make_async_copy(
            tok_hbm.at[idx_v.at[k]], rows_v.at[b], gsem[b]).start()

    def wait_gather(b):
        pltpu.make_async_copy(
            tok_hbm.at[idx_v.at[0]], rows_v.at[b], gsem[b]).wait()

    def start_store(k, b):
        pltpu.make_async_copy(
            rows_v.at[b], out_hbm.at[pl.ds(out_row0 + k * CHUNK, CHUNK)],
            ssem[b]).start()

    def wait_store(b):
        pltpu.make_async_copy(
            rows_v.at[b], out_hbm.at[pl.ds(out_row0, CHUNK)], ssem[b]).wait()

    start_gather(0, 0)
    start_gather(1, 1)
    for k in (0, 1):
        start_gather(k + 2, k + 2)
        wait_gather(k)
        _add_pos(rows_v, pos2_v, k, k)
        start_store(k, k)

    def group(g, carry):
        for j in range(NBUF):
            k = 2 + g * NBUF + j
            b = (2 + j) % NBUF
            wait_store(j)
            start_gather(k + 2, j)
            wait_gather(b)
            _add_pos(rows_v, pos2_v, b, k)
            start_store(k, b)
        return carry

    lax.fori_loop(0, (N_CHUNKS - NBUF) // NBUF, group, 0)

    for k in (N_CHUNKS - 2, N_CHUNKS - 1):
        b = k % NBUF
        wait_store(b - 2)
        wait_gather(b)
        _add_pos(rows_v, pos2_v, b, k)
        start_store(k, b)
    wait_store(NBUF - 2)
    wait_store(NBUF - 1)


def kernel(input_ids, token_table, pos_table):
    ids = input_ids.reshape(TOTAL // CHUNK, CHUNK).astype(jnp.int32)
    tok = token_table.astype(jnp.float32)
    pos = pos_table.astype(jnp.float32)

    mesh = plsc.VectorSubcoreMesh(core_axis_name="c", subcore_axis_name="s")
    run = pl.kernel(
        _kernel_body,
        out_type=jax.ShapeDtypeStruct((TOTAL, D), jnp.float32),
        mesh=mesh,
        compiler_params=pltpu.CompilerParams(use_tc_tiling_on_sc=False),
        scratch_types=[
            pltpu.VMEM((N_CHUNKS, CHUNK), jnp.int32),
            pltpu.VMEM((2 * SEQ, D), jnp.float32),
            pltpu.VMEM((NBUF, CHUNK, D), jnp.float32),
            pltpu.SemaphoreType.DMA, pltpu.SemaphoreType.DMA,
            pltpu.SemaphoreType.DMA, pltpu.SemaphoreType.DMA,
            pltpu.SemaphoreType.DMA, pltpu.SemaphoreType.DMA,
            pltpu.SemaphoreType.DMA, pltpu.SemaphoreType.DMA,
        ],
    )
    out = run(ids, tok, pos)
    return out.reshape(BATCH, SEQ, D)
